# Initial kernel scaffold; baseline (speedup 1.0000x reference)
import jax
import jax.numpy as jnp
from jax.experimental import pallas as pl


def kernel(x, W_qkv, W_proj, b_proj):
    raise NotImplementedError("write your pallas kernel here")


# speedup vs baseline: 24.3671x; 24.3671x over previous
import numpy as np
import jax
import jax.numpy as jnp
from jax.experimental import pallas as pl
from jax.experimental.pallas import tpu as pltpu

_B, _N, _DIM, _H = 16, 197, 768, 12
_d = _DIM // _H
_M = 25
_KVT = 25
_GRID_HW = 14
_POOL = 5
_SCALE = _d ** -0.5
_HIGH = jax.lax.Precision.HIGHEST
_BF = jnp.bfloat16


def _dot(a, b, dims):
    return jax.lax.dot_general(a, b, (dims, ((), ())),
                               preferred_element_type=jnp.float32)


def _pool_mat() -> np.ndarray:
    P = np.zeros((_POOL, _GRID_HW), np.float32)
    for i in range(_POOL):
        s = (i * _GRID_HW) // _POOL
        e = -((-(i + 1) * _GRID_HW) // _POOL)
        P[i, s:e] = 1.0 / (e - s)
    A = np.einsum('ph,qw->pqhw', P, P).reshape(_M, _GRID_HW * _GRID_HW)
    return np.concatenate([A, np.zeros((_M, 1), np.float32)], axis=1)


def _qkv_kernel(x_ref, w_ref, o_ref):
    x = x_ref[0]
    for j in range(3 * _H):
        o_ref[0, j] = _dot(x, w_ref[j], ((1,), (1,)))


def _attn_kernel(q_ref, k_ref, v_ref, a_ref, wp_ref, b_ref, o_ref):
    h = pl.program_id(1)
    q = q_ref[0, 0]
    qb = q.astype(_BF)
    kb = k_ref[0, 0].astype(_BF)
    vb = v_ref[0, 0].astype(_BF)

    r = jax.lax.dot_general(a_ref[...], q, (((1,), (0,)), ((), ())),
                            precision=_HIGH,
                            preferred_element_type=jnp.float32)
    rb = r.astype(_BF)
    rk = _dot(rb, kb, ((1,), (1,)))
    s_rk = rk * _SCALE
    s_rk = s_rk - jnp.max(s_rk, axis=1, keepdims=True)
    e_rk = jnp.exp(s_rk)
    agent_p = e_rk / jnp.sum(e_rk, axis=1, keepdims=True)
    agent_value = _dot(agent_p.astype(_BF), vb, ((1,), (0,)))

    iota_n = jax.lax.broadcasted_iota(jnp.int32, (_M, _N), 1)

    def body(_, carry):
        cur, mask = carry
        rowmax = jnp.max(cur, axis=1, keepdims=True)
        idx = jnp.min(jnp.where(cur == rowmax, iota_n, _N), axis=1,
                      keepdims=True)
        onehot = (iota_n == idx).astype(jnp.float32)
        return jnp.where(onehot > 0.5, -jnp.inf, cur), jnp.maximum(mask, onehot)

    _, keymask = jax.lax.fori_loop(
        0, _KVT, body, (rk, jnp.zeros((_M, _N), jnp.float32)))

    gate = _dot(rb, qb, ((1,), (1,)))
    iota_m = jax.lax.broadcasted_iota(jnp.int32, (_M, _N), 0)
    colmax = jnp.max(gate, axis=0, keepdims=True)
    eidx = jnp.min(jnp.where(gate == colmax, iota_m, _M), axis=0,
                   keepdims=True)
    onehot_e = (iota_m == eidx).astype(_BF)
    qmask = _dot(onehot_e, keymask.astype(_BF), ((0,), (0,)))
    sel = qmask > 0.5

    al = _SCALE * _dot(qb, rb, ((1,), (1,)))
    s = _SCALE * _dot(qb, kb, ((1,), (1,)))
    s_m = jnp.where(sel, s, -jnp.inf)
    mx = jnp.maximum(jnp.max(al, axis=1, keepdims=True),
                     jnp.max(s_m, axis=1, keepdims=True))
    e_a = jnp.exp(al - mx)
    e_s = jnp.where(sel, jnp.exp(s - mx), 0.0)
    denom = (jnp.sum(e_a, axis=1, keepdims=True)
             + jnp.sum(e_s, axis=1, keepdims=True))
    out64 = (_dot((e_a / denom).astype(_BF), agent_value.astype(_BF),
                  ((1,), (0,)))
             + _dot((e_s / denom).astype(_BF), vb, ((1,), (0,))))

    partial = _dot(out64.astype(_BF), wp_ref[0], ((1,), (0,)))

    @pl.when(h == 0)
    def _():
        o_ref[0] = partial + b_ref[...]

    @pl.when(h != 0)
    def _():
        o_ref[0] += partial


def kernel(x, W_qkv, W_proj, b_proj):
    A = jnp.asarray(_pool_mat())
    Wq = W_qkv.reshape(3 * _H, _d, _DIM).astype(_BF)
    Wp = W_proj.T.reshape(_H, _d, _DIM).astype(_BF)

    qkv = pl.pallas_call(
        _qkv_kernel,
        grid=(_B,),
        in_specs=[
            pl.BlockSpec((1, _N, _DIM), lambda b: (b, 0, 0)),
            pl.BlockSpec((3 * _H, _d, _DIM), lambda b: (0, 0, 0)),
        ],
        out_specs=pl.BlockSpec((1, 3 * _H, _N, _d), lambda b: (b, 0, 0, 0)),
        out_shape=jax.ShapeDtypeStruct((_B, 3 * _H, _N, _d), jnp.float32),
    )(x.astype(_BF), Wq)

    out = pl.pallas_call(
        _attn_kernel,
        grid=(_B, _H),
        in_specs=[
            pl.BlockSpec((1, 1, _N, _d), lambda b, h: (b, h, 0, 0)),
            pl.BlockSpec((1, 1, _N, _d), lambda b, h: (b, h + _H, 0, 0)),
            pl.BlockSpec((1, 1, _N, _d), lambda b, h: (b, h + 2 * _H, 0, 0)),
            pl.BlockSpec((_M, _N), lambda b, h: (0, 0)),
            pl.BlockSpec((1, _d, _DIM), lambda b, h: (h, 0, 0)),
            pl.BlockSpec((1, _DIM), lambda b, h: (0, 0)),
        ],
        out_specs=pl.BlockSpec((1, _N, _DIM), lambda b, h: (b, 0, 0)),
        out_shape=jax.ShapeDtypeStruct((_B, _N, _DIM), jnp.float32),
    )(qkv, qkv, qkv, A, Wp, b_proj.reshape(1, _DIM))

    return out


# speedup vs baseline: 63.4628x; 2.6044x over previous
import numpy as np
import jax
import jax.numpy as jnp
from jax.experimental import pallas as pl
from jax.experimental.pallas import tpu as pltpu

_B, _N, _DIM, _H = 16, 197, 768, 12
_d = _DIM // _H
_M = 25
_KVT = 25
_GRID_HW = 14
_POOL = 5
_SCALE = _d ** -0.5
_HIGH = jax.lax.Precision.HIGHEST
_BF = jnp.bfloat16
_ROWS = _B * _H * _M
_RBLK = 600


def _dot(a, b, dims):
    return jax.lax.dot_general(a, b, (dims, ((), ())),
                               preferred_element_type=jnp.float32)


def _pool_mat() -> np.ndarray:
    P = np.zeros((_POOL, _GRID_HW), np.float32)
    for i in range(_POOL):
        s = (i * _GRID_HW) // _POOL
        e = -((-(i + 1) * _GRID_HW) // _POOL)
        P[i, s:e] = 1.0 / (e - s)
    A = np.einsum('ph,qw->pqhw', P, P).reshape(_M, _GRID_HW * _GRID_HW)
    return np.concatenate([A, np.zeros((_M, 1), np.float32)], axis=1)


def _qkv_kernel(x_ref, w_ref, a_ref, o_ref, rk_ref):
    x = x_ref[0]
    qs = []
    for j in range(_H):
        qj = _dot(x, w_ref[j], ((1,), (1,)))
        o_ref[0, j] = qj
        qs.append(qj)
    rs = [jax.lax.dot_general(a_ref[...], qs[h], (((1,), (0,)), ((), ())),
                              precision=_HIGH,
                              preferred_element_type=jnp.float32).astype(_BF)
          for h in range(_H)]
    for h in range(_H):
        kj = _dot(x, w_ref[_H + h], ((1,), (1,)))
        o_ref[0, _H + h] = kj
        rk_ref[0, h] = _dot(rs[h], kj.astype(_BF), ((1,), (1,)))
    for j in range(2 * _H, 3 * _H):
        o_ref[0, j] = _dot(x, w_ref[j], ((1,), (1,)))


def _mask_kernel(rk_ref, m_ref):
    iota_n = jax.lax.broadcasted_iota(jnp.int32, (_RBLK, _N), 1)

    def body(_, carry):
        cur, mask = carry
        rowmax = jnp.max(cur, axis=1, keepdims=True)
        idx = jnp.min(jnp.where(cur == rowmax, iota_n, _N), axis=1,
                      keepdims=True)
        onehot = (iota_n == idx).astype(jnp.float32)
        return jnp.where(onehot > 0.5, -jnp.inf, cur), jnp.maximum(mask, onehot)

    _, keymask = jax.lax.fori_loop(
        0, _KVT, body, (rk_ref[...], jnp.zeros((_RBLK, _N), jnp.float32)))
    m_ref[...] = keymask


def _attn_kernel(q_ref, k_ref, v_ref, km_ref, a_ref, wp_ref, b_ref, o_ref):
    h = pl.program_id(1)
    q = q_ref[0, 0]
    qb = q.astype(_BF)
    kb = k_ref[0, 0].astype(_BF)
    vb = v_ref[0, 0].astype(_BF)
    keymask = km_ref[0, 0]

    r = jax.lax.dot_general(a_ref[...], q, (((1,), (0,)), ((), ())),
                            precision=_HIGH,
                            preferred_element_type=jnp.float32)
    rb = r.astype(_BF)
    rk = _dot(rb, kb, ((1,), (1,)))
    s_rk = rk * _SCALE
    s_rk = s_rk - jnp.max(s_rk, axis=1, keepdims=True)
    e_rk = jnp.exp(s_rk)
    agent_p = e_rk / jnp.sum(e_rk, axis=1, keepdims=True)
    agent_value = _dot(agent_p.astype(_BF), vb, ((1,), (0,)))

    gate = _dot(rb, qb, ((1,), (1,)))
    iota_m = jax.lax.broadcasted_iota(jnp.int32, (_M, _N), 0)
    colmax = jnp.max(gate, axis=0, keepdims=True)
    eidx = jnp.min(jnp.where(gate == colmax, iota_m, _M), axis=0,
                   keepdims=True)
    onehot_e = (iota_m == eidx).astype(_BF)
    qmask = _dot(onehot_e, keymask.astype(_BF), ((0,), (0,)))
    sel = qmask > 0.5

    al = _SCALE * _dot(qb, rb, ((1,), (1,)))
    s = _SCALE * _dot(qb, kb, ((1,), (1,)))
    s_m = jnp.where(sel, s, -jnp.inf)
    mx = jnp.maximum(jnp.max(al, axis=1, keepdims=True),
                     jnp.max(s_m, axis=1, keepdims=True))
    e_a = jnp.exp(al - mx)
    e_s = jnp.where(sel, jnp.exp(s - mx), 0.0)
    denom = (jnp.sum(e_a, axis=1, keepdims=True)
             + jnp.sum(e_s, axis=1, keepdims=True))
    out64 = (_dot((e_a / denom).astype(_BF), agent_value.astype(_BF),
                  ((1,), (0,)))
             + _dot((e_s / denom).astype(_BF), vb, ((1,), (0,))))

    partial = _dot(out64.astype(_BF), wp_ref[0], ((1,), (0,)))

    @pl.when(h == 0)
    def _():
        o_ref[0] = partial + b_ref[...]

    @pl.when(h != 0)
    def _():
        o_ref[0] += partial


def kernel(x, W_qkv, W_proj, b_proj):
    A = jnp.asarray(_pool_mat())
    Wq = W_qkv.reshape(3 * _H, _d, _DIM).astype(_BF)
    Wp = W_proj.T.reshape(_H, _d, _DIM).astype(_BF)

    qkv, rk = pl.pallas_call(
        _qkv_kernel,
        grid=(_B,),
        in_specs=[
            pl.BlockSpec((1, _N, _DIM), lambda b: (b, 0, 0)),
            pl.BlockSpec((3 * _H, _d, _DIM), lambda b: (0, 0, 0)),
            pl.BlockSpec((_M, _N), lambda b: (0, 0)),
        ],
        out_specs=[
            pl.BlockSpec((1, 3 * _H, _N, _d), lambda b: (b, 0, 0, 0)),
            pl.BlockSpec((1, _H, _M, _N), lambda b: (b, 0, 0, 0)),
        ],
        out_shape=[
            jax.ShapeDtypeStruct((_B, 3 * _H, _N, _d), jnp.float32),
            jax.ShapeDtypeStruct((_B, _H, _M, _N), jnp.float32),
        ],
    )(x.astype(_BF), Wq, A)

    keymask = pl.pallas_call(
        _mask_kernel,
        grid=(_ROWS // _RBLK,),
        in_specs=[pl.BlockSpec((_RBLK, _N), lambda i: (i, 0))],
        out_specs=pl.BlockSpec((_RBLK, _N), lambda i: (i, 0)),
        out_shape=jax.ShapeDtypeStruct((_ROWS, _N), jnp.float32),
    )(rk.reshape(_ROWS, _N))

    out = pl.pallas_call(
        _attn_kernel,
        grid=(_B, _H),
        in_specs=[
            pl.BlockSpec((1, 1, _N, _d), lambda b, h: (b, h, 0, 0)),
            pl.BlockSpec((1, 1, _N, _d), lambda b, h: (b, h + _H, 0, 0)),
            pl.BlockSpec((1, 1, _N, _d), lambda b, h: (b, h + 2 * _H, 0, 0)),
            pl.BlockSpec((1, 1, _M, _N), lambda b, h: (b, h, 0, 0)),
            pl.BlockSpec((_M, _N), lambda b, h: (0, 0)),
            pl.BlockSpec((1, _d, _DIM), lambda b, h: (h, 0, 0)),
            pl.BlockSpec((1, _DIM), lambda b, h: (0, 0)),
        ],
        out_specs=pl.BlockSpec((1, _N, _DIM), lambda b, h: (b, 0, 0)),
        out_shape=jax.ShapeDtypeStruct((_B, _N, _DIM), jnp.float32),
    )(qkv, qkv, qkv, keymask.reshape(_B, _H, _M, _N), A, Wp,
      b_proj.reshape(1, _DIM))

    return out


# speedup vs baseline: 108.3956x; 1.7080x over previous
import numpy as np
import jax
import jax.numpy as jnp
from jax.experimental import pallas as pl
from jax.experimental.pallas import tpu as pltpu

_B, _N, _DIM, _H = 16, 197, 768, 12
_d = _DIM // _H
_M = 25
_KVT = 25
_GRID_HW = 14
_POOL = 5
_SCALE = _d ** -0.5
_HIGH = jax.lax.Precision.HIGHEST
_BF = jnp.bfloat16
_NEG = float('-inf')


def _dot(a, b, dims):
    return jax.lax.dot_general(a, b, (dims, ((), ())),
                               preferred_element_type=jnp.float32)


def _pool_mat() -> np.ndarray:
    P = np.zeros((_POOL, _GRID_HW), np.float32)
    for i in range(_POOL):
        s = (i * _GRID_HW) // _POOL
        e = -((-(i + 1) * _GRID_HW) // _POOL)
        P[i, s:e] = 1.0 / (e - s)
    A = np.einsum('ph,qw->pqhw', P, P).reshape(_M, _GRID_HW * _GRID_HW)
    return np.concatenate([A, np.zeros((_M, 1), np.float32)], axis=1)


def _fused_kernel(x_ref, w_ref, a_ref, wp_ref, b_ref, o_ref):
    x = x_ref[0]
    q_full = _dot(x, w_ref[0], ((1,), (1,)))
    k_full = _dot(x, w_ref[1], ((1,), (1,)))
    v_full = _dot(x, w_ref[2], ((1,), (1,)))
    qb_full = q_full.astype(_BF)
    kb_full = k_full.astype(_BF)
    vb_full = v_full.astype(_BF)

    r_cat = jax.lax.dot_general(a_ref[...], q_full, (((1,), (0,)), ((), ())),
                                precision=_HIGH,
                                preferred_element_type=jnp.float32)
    rb_cat = r_cat.astype(_BF)

    qb = [qb_full[:, h * _d:(h + 1) * _d] for h in range(_H)]
    kb = [kb_full[:, h * _d:(h + 1) * _d] for h in range(_H)]
    vb = [vb_full[:, h * _d:(h + 1) * _d] for h in range(_H)]
    rb = [rb_cat[:, h * _d:(h + 1) * _d] for h in range(_H)]

    rk = [_dot(rb[h], kb[h], ((1,), (1,))) for h in range(_H)]

    iota_n = jax.lax.broadcasted_iota(jnp.int32, (_M, _N), 1)

    def body(_, curs):
        new = []
        for cur in curs:
            rowmax = jnp.max(cur, axis=1, keepdims=True)
            idx = jnp.min(jnp.where(cur == rowmax, iota_n, _N), axis=1,
                          keepdims=True)
            new.append(jnp.where(iota_n == idx, _NEG, cur))
        return tuple(new)

    curs = jax.lax.fori_loop(0, _KVT, body, tuple(rk))
    keymask = [(curs[h] == _NEG).astype(_BF) for h in range(_H)]

    iota_m = jax.lax.broadcasted_iota(jnp.int32, (_M, _N), 0)
    out_acc = b_ref[...]
    for h in range(_H):
        s_rk = rk[h] * _SCALE
        s_rk = s_rk - jnp.max(s_rk, axis=1, keepdims=True)
        e_rk = jnp.exp(s_rk)
        agent_p = e_rk / jnp.sum(e_rk, axis=1, keepdims=True)
        agent_value = _dot(agent_p.astype(_BF), vb[h], ((1,), (0,)))

        gate = _dot(rb[h], qb[h], ((1,), (1,)))
        colmax = jnp.max(gate, axis=0, keepdims=True)
        eidx = jnp.min(jnp.where(gate == colmax, iota_m, _M), axis=0,
                       keepdims=True)
        onehot_e = (iota_m == eidx).astype(_BF)
        qmask = _dot(onehot_e, keymask[h], ((0,), (0,)))
        sel = qmask > 0.5

        al = _SCALE * _dot(qb[h], rb[h], ((1,), (1,)))
        s = _SCALE * _dot(qb[h], kb[h], ((1,), (1,)))
        s_m = jnp.where(sel, s, _NEG)
        mx = jnp.maximum(jnp.max(al, axis=1, keepdims=True),
                         jnp.max(s_m, axis=1, keepdims=True))
        e_a = jnp.exp(al - mx)
        e_s = jnp.where(sel, jnp.exp(s - mx), 0.0)
        denom = (jnp.sum(e_a, axis=1, keepdims=True)
                 + jnp.sum(e_s, axis=1, keepdims=True))
        out64 = (_dot((e_a / denom).astype(_BF), agent_value.astype(_BF),
                      ((1,), (0,)))
                 + _dot((e_s / denom).astype(_BF), vb[h], ((1,), (0,))))

        out_acc = out_acc + _dot(out64.astype(_BF),
                                 wp_ref[h * _d:(h + 1) * _d, :],
                                 ((1,), (0,)))
    o_ref[0] = out_acc


def kernel(x, W_qkv, W_proj, b_proj):
    A = jnp.asarray(_pool_mat())
    W3 = W_qkv.reshape(3, _DIM, _DIM).astype(_BF)
    Wp = W_proj.T.astype(_BF)

    out = pl.pallas_call(
        _fused_kernel,
        grid=(_B,),
        in_specs=[
            pl.BlockSpec((1, _N, _DIM), lambda b: (b, 0, 0)),
            pl.BlockSpec((3, _DIM, _DIM), lambda b: (0, 0, 0)),
            pl.BlockSpec((_M, _N), lambda b: (0, 0)),
            pl.BlockSpec((_DIM, _DIM), lambda b: (0, 0)),
            pl.BlockSpec((1, _DIM), lambda b: (0, 0)),
        ],
        out_specs=pl.BlockSpec((1, _N, _DIM), lambda b: (b, 0, 0)),
        out_shape=jax.ShapeDtypeStruct((_B, _N, _DIM), jnp.float32),
    )(x.astype(_BF), W3, A, Wp, b_proj.reshape(1, _DIM))

    return out


# speedup vs baseline: 108.6317x; 1.0022x over previous
import numpy as np
import jax
import jax.numpy as jnp
from jax.experimental import pallas as pl
from jax.experimental.pallas import tpu as pltpu

_B, _N, _DIM, _H = 16, 197, 768, 12
_d = _DIM // _H
_M = 25
_KVT = 25
_GRID_HW = 14
_POOL = 5
_SCALE = _d ** -0.5
_HIGH = jax.lax.Precision.HIGHEST
_BF = jnp.bfloat16
_NEG = float('-inf')


def _dot(a, b, dims):
    return jax.lax.dot_general(a, b, (dims, ((), ())),
                               preferred_element_type=jnp.float32)


def _pool_mat() -> np.ndarray:
    P = np.zeros((_POOL, _GRID_HW), np.float32)
    for i in range(_POOL):
        s = (i * _GRID_HW) // _POOL
        e = -((-(i + 1) * _GRID_HW) // _POOL)
        P[i, s:e] = 1.0 / (e - s)
    A = np.einsum('ph,qw->pqhw', P, P).reshape(_M, _GRID_HW * _GRID_HW)
    return np.concatenate([A, np.zeros((_M, 1), np.float32)], axis=1)


def _fused_kernel(x_ref, w_ref, a_ref, wp_ref, b_ref, o_ref):
    x = x_ref[0]
    q_full = _dot(x, w_ref[0], ((1,), (1,)))
    k_full = _dot(x, w_ref[1], ((1,), (1,)))
    v_full = _dot(x, w_ref[2], ((1,), (1,)))
    qb_full = q_full.astype(_BF)
    kb_full = k_full.astype(_BF)
    vb_full = v_full.astype(_BF)

    r_cat = jax.lax.dot_general(a_ref[...], q_full, (((1,), (0,)), ((), ())),
                                precision=_HIGH,
                                preferred_element_type=jnp.float32)
    rb_cat = r_cat.astype(_BF)

    qb = [qb_full[:, h * _d:(h + 1) * _d] for h in range(_H)]
    kb = [kb_full[:, h * _d:(h + 1) * _d] for h in range(_H)]
    vb = [vb_full[:, h * _d:(h + 1) * _d] for h in range(_H)]
    rb = [rb_cat[:, h * _d:(h + 1) * _d] for h in range(_H)]

    rk = [_dot(rb[h], kb[h], ((1,), (1,))) for h in range(_H)]

    iota_n = jax.lax.broadcasted_iota(jnp.int32, (_M, _N), 1)

    def body(_, curs):
        new = []
        for cur in curs:
            rowmax = jnp.max(cur, axis=1, keepdims=True)
            idx = jnp.min(jnp.where(cur == rowmax, iota_n, _N), axis=1,
                          keepdims=True)
            new.append(jnp.where(iota_n == idx, _NEG, cur))
        return tuple(new)

    curs = jax.lax.fori_loop(0, _KVT, body, tuple(rk))
    keymask = [(curs[h] == _NEG).astype(_BF) for h in range(_H)]

    iota_m = jax.lax.broadcasted_iota(jnp.int32, (_M, _N), 0)
    out_acc = b_ref[...]
    for h in range(_H):
        s_rk = rk[h] * _SCALE
        s_rk = s_rk - jnp.max(s_rk, axis=1, keepdims=True)
        e_rk = jnp.exp(s_rk)
        agent_p = e_rk / jnp.sum(e_rk, axis=1, keepdims=True)
        agent_value = _dot(agent_p.astype(_BF), vb[h], ((1,), (0,)))

        gate = _dot(rb[h], qb[h], ((1,), (1,)))
        colmax = jnp.max(gate, axis=0, keepdims=True)
        eidx = jnp.min(jnp.where(gate == colmax, iota_m, _M), axis=0,
                       keepdims=True)
        onehot_e = (iota_m == eidx).astype(_BF)
        qmask = _dot(onehot_e, keymask[h], ((0,), (0,)))
        sel = qmask > 0.5

        al = _SCALE * _dot(qb[h], rb[h], ((1,), (1,)))
        s = _SCALE * _dot(qb[h], kb[h], ((1,), (1,)))
        s_m = jnp.where(sel, s, _NEG)
        mx = jnp.maximum(jnp.max(al, axis=1, keepdims=True),
                         jnp.max(s_m, axis=1, keepdims=True))
        e_a = jnp.exp(al - mx)
        e_s = jnp.exp(s_m - mx)
        denom = (jnp.sum(e_a, axis=1, keepdims=True)
                 + jnp.sum(e_s, axis=1, keepdims=True))
        out64 = (_dot((e_a / denom).astype(_BF), agent_value.astype(_BF),
                      ((1,), (0,)))
                 + _dot((e_s / denom).astype(_BF), vb[h], ((1,), (0,))))

        out_acc = out_acc + _dot(out64.astype(_BF),
                                 wp_ref[h * _d:(h + 1) * _d, :],
                                 ((1,), (0,)))
    o_ref[0] = out_acc


def kernel(x, W_qkv, W_proj, b_proj):
    A = jnp.asarray(_pool_mat())
    W3 = W_qkv.reshape(3, _DIM, _DIM).astype(_BF)
    Wp = W_proj.T.astype(_BF)

    out = pl.pallas_call(
        _fused_kernel,
        grid=(_B,),
        in_specs=[
            pl.BlockSpec((1, _N, _DIM), lambda b: (b, 0, 0)),
            pl.BlockSpec((3, _DIM, _DIM), lambda b: (0, 0, 0)),
            pl.BlockSpec((_M, _N), lambda b: (0, 0)),
            pl.BlockSpec((_DIM, _DIM), lambda b: (0, 0)),
            pl.BlockSpec((1, _DIM), lambda b: (0, 0)),
        ],
        out_specs=pl.BlockSpec((1, _N, _DIM), lambda b: (b, 0, 0)),
        out_shape=jax.ShapeDtypeStruct((_B, _N, _DIM), jnp.float32),
        compiler_params=pltpu.CompilerParams(
            dimension_semantics=("parallel",)),
    )(x.astype(_BF), W3, A, Wp, b_proj.reshape(1, _DIM))

    return out
